# baseline (speedup 1.0000x reference)
import functools

import jax
import jax.numpy as jnp
from jax import lax
from jax.experimental import pallas as pl
from jax.experimental.pallas import tpu as pltpu
from jax.experimental.pallas import tpu_sc as plsc

_B, _L = 1024, 200
_N = _B * _L
_HD = 64
_NC, _NS = 2, 16
_NW = _NC * _NS
_CHUNK = _N // _NW
_K = 128
_NBLK = _CHUNK // _K

_f32 = jnp.float32
_bf16 = jnp.bfloat16
_i32 = jnp.int32


def _fold_body(wi, wg, e9i, e9g, wte, wq, wtg, wg1, wse, wc,
               ii, gi, tei, qi, tgi, g1i, g2i, sei, bgi,
               o_ig, o_te, o_q, o_tg, o_g1, o_g2, o_se, o_idx):
    wcv = wc[...]

    def mm(a, b):
        return jnp.dot(a, b, preferred_element_type=_f32)

    a_int = mm(wi[...], wcv[0:21, :])
    a_gue = mm(wg[...], wcv[474:484, :])
    o_ig[...] = (mm(e9i[...], a_int) + mm(e9g[...], a_gue)).astype(_bf16)
    o_te[...] = mm(wte[...], wcv[21:42, :]).astype(_bf16)
    o_q[...] = mm(wq[...], wcv[42:63, :]).astype(_bf16)
    o_tg[...] = mm(wtg[...], wcv[63:84, :]).astype(_bf16)
    o_g1[...] = mm(wg1[...], wcv[85:200, :]).astype(_bf16)
    o_g2[...] = mm(wg1[...], wcv[200:315, :]).astype(_bf16)
    o_se[...] = mm(wse[...], wcv[315:415, :]).astype(_bf16)
    o_idx[...] = jnp.stack(
        [ii[...] * 3 + gi[...], tei[...], qi[...], tgi[...], g1i[...],
         g2i[...], sei[...], bgi[...]], axis=0)


def _fold(wi, wg, e9i, e9g, wte, wq, wtg, wg1, wse, wc, idx_arrays):
    n_te, n_q, n_tg, n_g1, n_se = (wte.shape[0], wq.shape[0], wtg.shape[0],
                                   wg1.shape[0], wse.shape[0])
    out_shape = (
        jax.ShapeDtypeStruct((9, _HD), _bf16),
        jax.ShapeDtypeStruct((n_te, _HD), _bf16),
        jax.ShapeDtypeStruct((n_q, _HD), _bf16),
        jax.ShapeDtypeStruct((n_tg, _HD), _bf16),
        jax.ShapeDtypeStruct((n_g1, _HD), _bf16),
        jax.ShapeDtypeStruct((n_g1, _HD), _bf16),
        jax.ShapeDtypeStruct((n_se, _HD), _bf16),
        jax.ShapeDtypeStruct((8, _N), _i32),
    )
    return pl.pallas_call(_fold_body, out_shape=out_shape)(
        wi, wg, e9i, e9g, wte, wq, wtg, wg1, wse, wc, *idx_arrays)


_PB = 8192


def _project_body(w_big, w50, o):
    o[...] = jnp.dot(w_big[...], w50[...],
                     preferred_element_type=_f32).astype(_bf16)


def _project(w_big, w50):
    v = w_big.shape[0]
    grid = (pl.cdiv(v, _PB),)
    return pl.pallas_call(
        _project_body,
        grid=grid,
        in_specs=[
            pl.BlockSpec((_PB, 50), lambda i: (i, 0)),
            pl.BlockSpec((50, _HD), lambda i: (0, 0)),
        ],
        out_specs=pl.BlockSpec((_PB, _HD), lambda i: (i, 0)),
        out_shape=jax.ShapeDtypeStruct((v, _HD), _bf16),
    )(w_big, w50)


def _sc_body(idx_hbm,
             t_ig, t_te, t_q, t_tg, t_g1, t_g2, t_se, t_big,
             s_hbm,
             idx_all, rows, obuf, sem_g0, sem_g1, sem_s0, sem_s1):
    cid = lax.axis_index("c")
    sid = lax.axis_index("s")
    wid = sid * _NC + cid
    base = wid * _CHUNK

    tables = (t_ig, t_te, t_q, t_tg, t_g1, t_g2, t_se, t_big)
    sems_g = (sem_g0, sem_g1)
    sems_s = (sem_s0, sem_s1)

    stage = pltpu.make_async_copy(
        idx_hbm.at[:, pl.ds(base, _CHUNK)], idx_all, sem_g0)
    stage.start()
    stage.wait()

    def gather_cp(blk, p, j):
        return pltpu.make_async_copy(
            tables[j].at[idx_all.at[j, pl.ds(blk * _K, _K)]], rows.at[p, j],
            sems_g[p])

    def start_gathers(blk, p):
        for j in range(8):
            gather_cp(blk, p, j).start()

    def wait_gathers(blk, p):
        for j in range(8):
            gather_cp(blk, p, j).wait()

    def scatter_cp(blk, p):
        return pltpu.make_async_copy(
            obuf.at[p], s_hbm.at[pl.ds(base + blk * _K, _K)], sems_s[p])

    def sum_rows(p):
        def tok(t, c2):
            for c in range(_HD // 32):
                sl = pl.ds(c * 32, 32)
                acc = rows[p, 0, t, sl]
                for j in range(1, 8):
                    acc = acc + rows[p, j, t, sl]
                obuf[p, t, sl] = acc
            return c2

        lax.fori_loop(0, _K, tok, 0, unroll=4)

    start_gathers(0, 0)

    def pair_body(i, carry):
        b0 = 2 * i
        b1 = b0 + 1

        wait_gathers(b0, 0)
        start_gathers(b1, 1)

        @pl.when(i > 0)
        def _():
            scatter_cp(b0 - 2, 0).wait()

        sum_rows(0)
        scatter_cp(b0, 0).start()

        wait_gathers(b1, 1)

        @pl.when(i < _NBLK // 2 - 1)
        def _():
            start_gathers(b0 + 2, 0)

        @pl.when(i > 0)
        def _():
            scatter_cp(b1 - 2, 1).wait()

        sum_rows(1)
        scatter_cp(b1, 1).start()
        return carry

    lax.fori_loop(0, _NBLK // 2, pair_body, 0)
    scatter_cp(_NBLK - 2, 0).wait()
    scatter_cp(_NBLK - 1, 1).wait()


def _sc_gather_sum(idx_packed,
                   t_ig, t_te, t_q, t_tg, t_g1, t_g2, t_se, t_big):
    mesh = plsc.VectorSubcoreMesh(core_axis_name="c", subcore_axis_name="s")
    kern = pl.kernel(
        _sc_body,
        compiler_params=pltpu.CompilerParams(use_tc_tiling_on_sc=False),
        out_type=jax.ShapeDtypeStruct((_N, _HD), _bf16),
        mesh=mesh,
        scratch_types=[
            pltpu.VMEM((8, _CHUNK), _i32),
            pltpu.VMEM((2, 8, _K, _HD), _bf16),
            pltpu.VMEM((2, _K, _HD), _bf16),
            pltpu.SemaphoreType.DMA,
            pltpu.SemaphoreType.DMA,
            pltpu.SemaphoreType.DMA,
            pltpu.SemaphoreType.DMA,
        ],
    )
    return kern(idx_packed, t_ig, t_te, t_q, t_tg, t_g1, t_g2, t_se, t_big)


_TB = 2048


def _combine_body(s, sc, w12, b, o):
    o[...] = (s[...].astype(_f32) +
              jnp.dot(sc[...], w12[...], preferred_element_type=_f32) +
              b[...])


def _combine(s, scal, w12, b):
    grid = (_N // _TB,)
    return pl.pallas_call(
        _combine_body,
        grid=grid,
        in_specs=[
            pl.BlockSpec((_TB, _HD), lambda i: (i, 0)),
            pl.BlockSpec((_TB, 12), lambda i: (i, 0)),
            pl.BlockSpec((12, _HD), lambda i: (0, 0)),
            pl.BlockSpec((1, _HD), lambda i: (0, 0)),
        ],
        out_specs=pl.BlockSpec((_TB, _HD), lambda i: (i, 0)),
        out_shape=jax.ShapeDtypeStruct((_N, _HD), _f32),
    )(s, scal, w12, b)


def kernel(test, question, tag, correct, mask, interaction, duration,
           startTime, elapsedTime, test_group_one, test_group_two, serial,
           solved_count, correct_before, wrong_before, same_tag_solved_count,
           same_tag_correct_before, same_tag_wrong_before,
           item_correct_percent, user_correct_percent, current_correct_count,
           tag_group_one, tag_group_two, time_for_solve, guess_yn,
           guess_yn_user, guess_yn_test, guess_yn_serial, guess_yn_assessment,
           guess_yn_tag, guess_yn_day, guess_yn_group_one, guess_yn_group_two,
           correct_percent_group_one, correct_percent_group_two,
           correct_percent_serial, day_of_week, duration_user,
           item_difficulty, W_interaction, W_test, W_question, W_tag,
           W_test_group_one, W_serial, W_tag_group_two, W_guess, W_comb,
           b_comb):
    batch_size = interaction.shape[0]

    r9 = jnp.arange(9)
    e9i = (r9[:, None] // 3 == jnp.arange(3)[None, :]).astype(_f32)
    e9g = (r9[:, None] % 3 == jnp.arange(3)[None, :]).astype(_f32)

    flat = lambda a: a.reshape(_N)
    idx_arrays = (flat(interaction), flat(guess_yn), flat(test),
                  flat(question), flat(tag), flat(test_group_one),
                  flat(test_group_two), flat(serial), flat(tag_group_two))
    *folded, idx_packed = _fold(W_interaction, W_guess, e9i, e9g, W_test,
                                W_question, W_tag, W_test_group_one,
                                W_serial, W_comb, idx_arrays)
    w50 = lax.slice(W_comb, (422, 0), (472, _HD))
    t_big = _project(W_tag_group_two, w50)

    s_out = _sc_gather_sum(idx_packed, *folded, t_big)

    scal = jnp.stack(
        [duration, solved_count, correct_before, wrong_before,
         same_tag_solved_count, same_tag_correct_before,
         same_tag_wrong_before, current_correct_count, time_for_solve,
         user_correct_percent, day_of_week.astype(_f32), item_difficulty],
        axis=-1).reshape(_N, 12)

    w12 = W_comb[jnp.array([84, 415, 416, 417, 418, 419, 420, 421,
                            472, 473, 484, 485]), :]
    x = _combine(s_out, scal, w12, b_comb.reshape(1, _HD))
    return (x.reshape(_B, _L, _HD), batch_size)


# speedup vs baseline: 1.7227x; 1.7227x over previous
import functools

import jax
import jax.numpy as jnp
from jax import lax
from jax.experimental import pallas as pl
from jax.experimental.pallas import tpu as pltpu
from jax.experimental.pallas import tpu_sc as plsc

_B, _L = 1024, 200
_N = _B * _L
_HD = 64
_NC, _NS = 2, 16
_NW = _NC * _NS
_CHUNK = _N // _NW
_K = 128
_NBLK = _CHUNK // _K

_f32 = jnp.float32
_bf16 = jnp.bfloat16
_i32 = jnp.int32


def _fold_body(wi, wg, e9i, e9g, wte, wq, wtg, wg1, wse, wc,
               ii, gi, tei, qi, tgi, g1i, g2i, sei, bgi,
               o_igt, o_te, o_q, o_g1, o_g2, o_se, o_idx):
    wcv = wc[...]

    def mm(a, b):
        return jnp.dot(a, b, preferred_element_type=_f32)

    a_int = mm(wi[...], wcv[0:21, :])
    a_gue = mm(wg[...], wcv[474:484, :])
    a_ig = mm(e9i[...], a_int) + mm(e9g[...], a_gue)
    a_tag = mm(wtg[...], wcv[63:84, :])
    nt = a_tag.shape[0]
    o_igt[...] = (a_ig[:, None, :] + a_tag[None, :, :]).reshape(9 * nt, _HD)
    o_te[...] = mm(wte[...], wcv[21:42, :])
    o_q[...] = mm(wq[...], wcv[42:63, :])
    o_g1[...] = mm(wg1[...], wcv[85:200, :])
    o_g2[...] = mm(wg1[...], wcv[200:315, :])
    o_se[...] = mm(wse[...], wcv[315:415, :])
    o_idx[...] = jnp.stack(
        [(ii[...] * 3 + gi[...]) * nt + tgi[...], tei[...], qi[...],
         g1i[...], g2i[...], sei[...], bgi[...]], axis=0)


def _fold(wi, wg, e9i, e9g, wte, wq, wtg, wg1, wse, wc, idx_arrays):
    n_te, n_q, n_tg, n_g1, n_se = (wte.shape[0], wq.shape[0], wtg.shape[0],
                                   wg1.shape[0], wse.shape[0])
    out_shape = (
        jax.ShapeDtypeStruct((9 * n_tg, _HD), _f32),
        jax.ShapeDtypeStruct((n_te, _HD), _f32),
        jax.ShapeDtypeStruct((n_q, _HD), _f32),
        jax.ShapeDtypeStruct((n_g1, _HD), _f32),
        jax.ShapeDtypeStruct((n_g1, _HD), _f32),
        jax.ShapeDtypeStruct((n_se, _HD), _f32),
        jax.ShapeDtypeStruct((7, _N), _i32),
    )
    return pl.pallas_call(_fold_body, out_shape=out_shape)(
        wi, wg, e9i, e9g, wte, wq, wtg, wg1, wse, wc, *idx_arrays)


_PB = 8192


def _project_body(w_big, w50, o):
    o[...] = jnp.dot(w_big[...], w50[...], preferred_element_type=_f32)


def _project(w_big, w50):
    v = w_big.shape[0]
    grid = (pl.cdiv(v, _PB),)
    return pl.pallas_call(
        _project_body,
        grid=grid,
        in_specs=[
            pl.BlockSpec((_PB, 50), lambda i: (i, 0)),
            pl.BlockSpec((50, _HD), lambda i: (0, 0)),
        ],
        out_specs=pl.BlockSpec((_PB, _HD), lambda i: (i, 0)),
        out_shape=jax.ShapeDtypeStruct((v, _HD), _f32),
    )(w_big, w50)


def _sc_body(idx_hbm,
             t_igt, t_te, t_q, t_g1, t_g2, t_se, t_big,
             s_hbm,
             idx_all, rows, sem_g, sem_s):
    cid = lax.axis_index("c")
    sid = lax.axis_index("s")
    wid = sid * _NC + cid
    base = wid * _CHUNK

    tables = (t_igt, t_te, t_q, t_g1, t_g2, t_se, t_big)

    stage = pltpu.make_async_copy(
        idx_hbm.at[:, pl.ds(base, _CHUNK)], idx_all, sem_g)
    stage.start()
    stage.wait()

    def scatter_cp(blk):
        return pltpu.make_async_copy(
            rows.at[0], s_hbm.at[pl.ds(base + blk * _K, _K)], sem_s)

    def blk_body(blk, carry):
        off = blk * _K

        @pl.when(blk > 0)
        def _():
            scatter_cp(blk - 1).wait()

        cps = []
        for j in range(7):
            cp = pltpu.make_async_copy(
                tables[j].at[idx_all.at[j, pl.ds(off, _K)]], rows.at[j],
                sem_g)
            cp.start()
            cps.append(cp)
        for cp in cps:
            cp.wait()

        def tok(t, c2):
            for c in range(_HD // 16):
                sl = pl.ds(c * 16, 16)
                acc = rows[0, t, sl]
                for j in range(1, 7):
                    acc = acc + rows[j, t, sl]
                rows[0, t, sl] = acc
            return c2

        lax.fori_loop(0, _K, tok, 0, unroll=4)

        scatter_cp(blk).start()
        return carry

    lax.fori_loop(0, _NBLK, blk_body, 0)
    scatter_cp(_NBLK - 1).wait()


def _sc_gather_sum(idx_packed,
                   t_igt, t_te, t_q, t_g1, t_g2, t_se, t_big):
    mesh = plsc.VectorSubcoreMesh(core_axis_name="c", subcore_axis_name="s")
    kern = pl.kernel(
        _sc_body,
        compiler_params=pltpu.CompilerParams(use_tc_tiling_on_sc=False),
        out_type=jax.ShapeDtypeStruct((_N, _HD), _f32),
        mesh=mesh,
        scratch_types=[
            pltpu.VMEM((7, _CHUNK), _i32),
            pltpu.VMEM((7, _K, _HD), _f32),
            pltpu.SemaphoreType.DMA,
            pltpu.SemaphoreType.DMA,
        ],
    )
    return kern(idx_packed, t_igt, t_te, t_q, t_g1, t_g2, t_se, t_big)


_TB = 2048


def _combine_body(s, sc, w12, b, o):
    o[...] = s[...] + jnp.dot(sc[...], w12[...],
                              preferred_element_type=_f32) + b[...]


def _combine(s, scal, w12, b):
    grid = (_N // _TB,)
    return pl.pallas_call(
        _combine_body,
        grid=grid,
        in_specs=[
            pl.BlockSpec((_TB, _HD), lambda i: (i, 0)),
            pl.BlockSpec((_TB, 12), lambda i: (i, 0)),
            pl.BlockSpec((12, _HD), lambda i: (0, 0)),
            pl.BlockSpec((1, _HD), lambda i: (0, 0)),
        ],
        out_specs=pl.BlockSpec((_TB, _HD), lambda i: (i, 0)),
        out_shape=jax.ShapeDtypeStruct((_N, _HD), _f32),
    )(s, scal, w12, b)


def kernel(test, question, tag, correct, mask, interaction, duration,
           startTime, elapsedTime, test_group_one, test_group_two, serial,
           solved_count, correct_before, wrong_before, same_tag_solved_count,
           same_tag_correct_before, same_tag_wrong_before,
           item_correct_percent, user_correct_percent, current_correct_count,
           tag_group_one, tag_group_two, time_for_solve, guess_yn,
           guess_yn_user, guess_yn_test, guess_yn_serial, guess_yn_assessment,
           guess_yn_tag, guess_yn_day, guess_yn_group_one, guess_yn_group_two,
           correct_percent_group_one, correct_percent_group_two,
           correct_percent_serial, day_of_week, duration_user,
           item_difficulty, W_interaction, W_test, W_question, W_tag,
           W_test_group_one, W_serial, W_tag_group_two, W_guess, W_comb,
           b_comb):
    batch_size = interaction.shape[0]

    r9 = jnp.arange(9)
    e9i = (r9[:, None] // 3 == jnp.arange(3)[None, :]).astype(_f32)
    e9g = (r9[:, None] % 3 == jnp.arange(3)[None, :]).astype(_f32)

    flat = lambda a: a.reshape(_N)
    idx_arrays = (flat(interaction), flat(guess_yn), flat(test),
                  flat(question), flat(tag), flat(test_group_one),
                  flat(test_group_two), flat(serial), flat(tag_group_two))
    *folded, idx_packed = _fold(W_interaction, W_guess, e9i, e9g, W_test,
                                W_question, W_tag, W_test_group_one,
                                W_serial, W_comb, idx_arrays)
    w50 = lax.slice(W_comb, (422, 0), (472, _HD))
    t_big = _project(W_tag_group_two, w50)

    s_out = _sc_gather_sum(idx_packed, *folded, t_big)

    scal = jnp.stack(
        [duration, solved_count, correct_before, wrong_before,
         same_tag_solved_count, same_tag_correct_before,
         same_tag_wrong_before, current_correct_count, time_for_solve,
         user_correct_percent, day_of_week.astype(_f32), item_difficulty],
        axis=-1).reshape(_N, 12)

    w12 = W_comb[jnp.array([84, 415, 416, 417, 418, 419, 420, 421,
                            472, 473, 484, 485]), :]
    x = _combine(s_out, scal, w12, b_comb.reshape(1, _HD))
    return (x.reshape(_B, _L, _HD), batch_size)
